# Initial kernel scaffold; baseline (speedup 1.0000x reference)
import jax
import jax.numpy as jnp
from jax.experimental import pallas as pl


def kernel(x, edge_index, mask, params):
    raise NotImplementedError("write your pallas kernel here")


# speedup vs baseline: 1.6344x; 1.6344x over previous
import functools
import jax
import jax.numpy as jnp
from jax.experimental import pallas as pl
from jax.experimental.pallas import tpu as pltpu


def _tail_kernel(h_ref, mask_ref, w1_ref, b1_ref, w2_ref, b2_ref, w3_ref,
                 b3_ref, out_ref, val_ref):
    h = h_ref[...]
    z = jnp.maximum(jnp.dot(h, w1_ref[...], preferred_element_type=jnp.float32)
                    + b1_ref[...][None, :], 0.0)
    z = jnp.maximum(jnp.dot(z, w2_ref[...], preferred_element_type=jnp.float32)
                    + b2_ref[...][None, :], 0.0)
    res = jnp.dot(z, w3_ref[...], preferred_element_type=jnp.float32)
    res = res + b3_ref[0]
    out_ref[...] = res[:, 0] * mask_ref[...]
    val_ref[...] = jnp.mean(res)[None, None]


def _mlp_tail(h, mask, m):
    n = h.shape[0]
    w1 = m['W1'][:, :3].T
    out, rmean = pl.pallas_call(
        _tail_kernel,
        out_shape=(
            jax.ShapeDtypeStruct((n,), jnp.float32),
            jax.ShapeDtypeStruct((1, 1), jnp.float32),
        ),
    )(h, mask, w1, m['b1'], m['W2'].T, m['b2'], m['W3'].T, m['b3'])
    return out, rmean[0, 0]


def kernel(x, edge_index, mask, params):
    n = x.shape[0]
    loop = jnp.arange(n, dtype=edge_index.dtype)
    src = jnp.concatenate([edge_index[0], loop])
    dst = jnp.concatenate([edge_index[1], loop])
    h = x
    nl = len(params['gat'])
    for i in range(nl):
        g = params['gat'][i]
        hW = h @ g['W'].T
        s_src = hW @ g['a_src']
        s_dst = hW @ g['a_dst']
        M = jax.nn.leaky_relu(jnp.max(s_src) + jnp.max(s_dst), 0.2)
        e = jax.nn.leaky_relu(s_src[src] + s_dst[dst], 0.2)
        w_ = jnp.exp(e - M)
        den = jax.ops.segment_sum(w_, dst, num_segments=n)
        num = jax.ops.segment_sum(w_[:, None] * hW[src], dst, num_segments=n)
        h = num / (den[:, None] + 1e-30) + g['b']
        if i < nl - 1:
            h = jax.nn.relu(h)
    p = params['mha']
    q = h @ p['Wq'].T + p['bq']
    k = h @ p['Wk'].T + p['bk']
    v = h @ p['Wv'].T + p['bv']
    outs = []
    for hh in range(3):
        qh, kh, vh = q[:, hh], k[:, hh], v[:, hh]
        kmax, kmin = jnp.max(kh), jnp.min(kh)
        mrow = jnp.where(qh >= 0, qh * kmax, qh * kmin)
        ex = jnp.exp(qh[:, None] * kh[None, :] - mrow[:, None])
        outs.append((ex @ vh) / ex.sum(1))
    h = jnp.stack(outs, axis=1) @ p['Wo'].T + p['bo']
    tc = params['tc']
    qt = h @ tc['Wq'].T
    kt = h @ tc['Wk'].T
    vt = h @ tc['Wv'].T
    sc = (qt[dst] * kt[src]).sum(-1) / jnp.sqrt(3.0)
    mt = (3.0 * jnp.max(jnp.abs(qt)) * jnp.max(jnp.abs(kt))) / jnp.sqrt(3.0)
    wt = jnp.exp(sc - mt)
    a0 = jax.ops.segment_sum(wt, dst, num_segments=n)
    a1 = jax.ops.segment_sum(wt[:, None] * vt[src], dst, num_segments=n)
    h = a1 / (a0[:, None] + 1e-30) + h @ tc['Ws'].T + tc['b']
    new_results, rmean = _mlp_tail(h, mask, params['mlp'])
    value = rmean * params['critic']['W'][0, 0] + params['critic']['b'][0]
    return new_results, value


# speedup vs baseline: 94.8311x; 58.0212x over previous
import functools
import jax
import jax.numpy as jnp
from jax import lax
from jax.experimental import pallas as pl
from jax.experimental.pallas import tpu as pltpu
from jax.experimental.pallas import tpu_sc as plsc

NT = 16
NPAD = 10240
ROWS_PT = NPAD // NT
EPAD = 655360
EPT = EPAD // NT
ECH = 4096
PADROW = 10200

_mesh = plsc.VectorSubcoreMesh(core_axis_name="c", subcore_axis_name="s",
                               num_cores=1)

_F32 = jnp.float32


def _zero_accs(a0, a1, a2, ad):
    z = jnp.zeros((16,), _F32)

    def zl(i, c):
        s = pl.ds(i * 16, 16)
        a0[s] = z
        a1[s] = z
        a2[s] = z
        ad[s] = z
        return c

    lax.fori_loop(0, NPAD // 16, zl, 0)


def _reduce_and_emit(w, a0, a1, a2, ad, sh, red, racc, rden, o0, o1, o2):
    rb = pl.ds(w * ROWS_PT, ROWS_PT)
    z = jnp.zeros((16,), _F32)

    for priv, oc in ((ad, None), (a0, o0), (a1, o1), (a2, o2)):
        pltpu.sync_copy(priv, sh.at[w])
        plsc.subcore_barrier()

        def zacc(i, c):
            racc[pl.ds(i * 16, 16)] = z
            return c

        def aacc(i, c):
            s = pl.ds(i * 16, 16)
            racc[s] = racc[s] + red[s]
            return c

        lax.fori_loop(0, ROWS_PT // 16, zacc, 0)
        for t in range(NT):
            pltpu.sync_copy(sh.at[t, rb], red)
            lax.fori_loop(0, ROWS_PT // 16, aacc, 0)

        if oc is None:
            def keep(i, c):
                s = pl.ds(i * 16, 16)
                rden[s] = jnp.maximum(racc[s], 1e-30)
                return c

            lax.fori_loop(0, ROWS_PT // 16, keep, 0)
        else:
            def dv(i, c):
                s = pl.ds(i * 16, 16)
                racc[s] = racc[s] / rden[s]
                return c

            lax.fori_loop(0, ROWS_PT // 16, dv, 0)
            pltpu.sync_copy(racc, oc.at[rb])
        plsc.subcore_barrier()


def _gat_body(ssrc_h, sdst_h, p0_h, p1_h, p2_h, src_h, dst_h, cons_h,
              o0, o1, o2,
              ssrc, sdst, p0, p1, p2, a0, a1, a2, ad, ebs, ebd, cbuf,
              red, racc, rden, sh):
    w = lax.axis_index("s")
    pltpu.sync_copy(ssrc_h, ssrc)
    pltpu.sync_copy(sdst_h, sdst)
    pltpu.sync_copy(p0_h, p0)
    pltpu.sync_copy(p1_h, p1)
    pltpu.sync_copy(p2_h, p2)
    pltpu.sync_copy(cons_h, cbuf)
    m = cbuf[pl.ds(0, 16)]
    _zero_accs(a0, a1, a2, ad)
    ebase = w * EPT
    for c in range(EPT // ECH):
        pltpu.sync_copy(src_h.at[pl.ds(ebase + c * ECH, ECH)], ebs)
        pltpu.sync_copy(dst_h.at[pl.ds(ebase + c * ECH, ECH)], ebd)

        def el(j, cc):
            s = pl.ds(j * 16, 16)
            si = ebs[s]
            di = ebd[s]
            e = plsc.load_gather(ssrc, [si]) + plsc.load_gather(sdst, [di])
            e = jnp.maximum(e, e * 0.2)
            wt = jnp.exp(e - m)
            plsc.addupdate_scatter(ad, [di], wt)
            plsc.addupdate_scatter(a0, [di], wt * plsc.load_gather(p0, [si]))
            plsc.addupdate_scatter(a1, [di], wt * plsc.load_gather(p1, [si]))
            plsc.addupdate_scatter(a2, [di], wt * plsc.load_gather(p2, [si]))
            return cc

        lax.fori_loop(0, ECH // 16, el, 0)
    _reduce_and_emit(w, a0, a1, a2, ad, sh, red, racc, rden, o0, o1, o2)


def _tc_body(u0_h, u1_h, u2_h, h0_h, h1_h, h2_h, src_h, dst_h, cons_h,
             o0, o1, o2,
             u0, u1, u2, t0, t1, t2, a0, a1, a2, ad, ebs, ebd, cbuf,
             red, racc, rden, sh):
    w = lax.axis_index("s")
    pltpu.sync_copy(u0_h, u0)
    pltpu.sync_copy(u1_h, u1)
    pltpu.sync_copy(u2_h, u2)
    pltpu.sync_copy(h0_h, t0)
    pltpu.sync_copy(h1_h, t1)
    pltpu.sync_copy(h2_h, t2)
    pltpu.sync_copy(cons_h, cbuf)
    m = cbuf[pl.ds(0, 16)]
    wv = [[cbuf[pl.ds(16 * (1 + 3 * i + j), 16)] for j in range(3)]
          for i in range(3)]
    _zero_accs(a0, a1, a2, ad)
    ebase = w * EPT
    for c in range(EPT // ECH):
        pltpu.sync_copy(src_h.at[pl.ds(ebase + c * ECH, ECH)], ebs)
        pltpu.sync_copy(dst_h.at[pl.ds(ebase + c * ECH, ECH)], ebd)

        def el(j, cc):
            s = pl.ds(j * 16, 16)
            si = ebs[s]
            di = ebd[s]
            hg0 = plsc.load_gather(t0, [si])
            hg1 = plsc.load_gather(t1, [si])
            hg2 = plsc.load_gather(t2, [si])
            e = (plsc.load_gather(u0, [di]) * hg0 +
                 plsc.load_gather(u1, [di]) * hg1 +
                 plsc.load_gather(u2, [di]) * hg2)
            wt = jnp.exp(e - m)
            plsc.addupdate_scatter(ad, [di], wt)
            v0 = hg0 * wv[0][0] + hg1 * wv[0][1] + hg2 * wv[0][2]
            v1 = hg0 * wv[1][0] + hg1 * wv[1][1] + hg2 * wv[1][2]
            v2 = hg0 * wv[2][0] + hg1 * wv[2][1] + hg2 * wv[2][2]
            plsc.addupdate_scatter(a0, [di], wt * v0)
            plsc.addupdate_scatter(a1, [di], wt * v1)
            plsc.addupdate_scatter(a2, [di], wt * v2)
            return cc

        lax.fori_loop(0, ECH // 16, el, 0)
    _reduce_and_emit(w, a0, a1, a2, ad, sh, red, racc, rden, o0, o1, o2)


def _edge_pass_scratch():
    return [
        pltpu.VMEM((NPAD,), _F32),
        pltpu.VMEM((NPAD,), _F32),
        pltpu.VMEM((NPAD,), _F32),
        pltpu.VMEM((NPAD,), _F32),
        pltpu.VMEM((NPAD,), _F32),
        pltpu.VMEM((NPAD,), _F32),
        pltpu.VMEM((NPAD,), _F32),
        pltpu.VMEM((NPAD,), _F32),
        pltpu.VMEM((NPAD,), _F32),
        pltpu.VMEM((ECH,), jnp.int32),
        pltpu.VMEM((ECH,), jnp.int32),
        pltpu.VMEM((160,), _F32),
        pltpu.VMEM((ROWS_PT,), _F32),
        pltpu.VMEM((ROWS_PT,), _F32),
        pltpu.VMEM((ROWS_PT,), _F32),
        pltpu.VMEM_SHARED((NT, NPAD), _F32),
    ]


def _tc_scratch():
    sc = _edge_pass_scratch()
    return sc[:5] + [pltpu.VMEM((NPAD,), _F32)] + sc[5:]


_OUT3 = (jax.ShapeDtypeStruct((NPAD,), _F32),) * 3

_gat_pass = functools.partial(
    pl.kernel, out_type=_OUT3, mesh=_mesh,
    compiler_params=pltpu.CompilerParams(needs_layout_passes=False),
    scratch_types=_edge_pass_scratch())(_gat_body)

_tc_pass = functools.partial(
    pl.kernel, out_type=_OUT3, mesh=_mesh,
    compiler_params=pltpu.CompilerParams(needs_layout_passes=False),
    scratch_types=_tc_scratch())(_tc_body)


def _tail_kernel(h_ref, mask_ref, w1_ref, b1_ref, w2_ref, b2_ref, w3_ref,
                 b3_ref, out_ref, val_ref):
    h = h_ref[...]
    z = jnp.maximum(jnp.dot(h, w1_ref[...], preferred_element_type=_F32)
                    + b1_ref[...][None, :], 0.0)
    z = jnp.maximum(jnp.dot(z, w2_ref[...], preferred_element_type=_F32)
                    + b2_ref[...][None, :], 0.0)
    res = jnp.dot(z, w3_ref[...], preferred_element_type=_F32)
    res = res + b3_ref[0]
    out_ref[...] = res[:, 0] * mask_ref[...]
    val_ref[...] = jnp.mean(res)[None, None]


def _mlp_tail(h, mask, m):
    n = h.shape[0]
    w1 = m['W1'][:, :3].T
    out, rmean = pl.pallas_call(
        _tail_kernel,
        out_shape=(
            jax.ShapeDtypeStruct((n,), _F32),
            jax.ShapeDtypeStruct((1, 1), _F32),
        ),
    )(h, mask, w1, m['b1'], m['W2'].T, m['b2'], m['W3'].T, m['b3'])
    return out, rmean[0, 0]


def _padn(v):
    return jnp.pad(v, (0, NPAD - v.shape[0]))


def kernel(x, edge_index, mask, params):
    n = x.shape[0]
    loop = jnp.arange(n, dtype=jnp.int32)
    nreal = edge_index.shape[1] + n
    npadE = EPAD - nreal
    srcp = jnp.concatenate([edge_index[0].astype(jnp.int32), loop,
                            jnp.zeros((npadE,), jnp.int32)])
    dstp = jnp.concatenate([edge_index[1].astype(jnp.int32), loop,
                            jnp.full((npadE,), PADROW, jnp.int32)])
    h = x
    nl = len(params['gat'])
    for i in range(nl):
        g = params['gat'][i]
        hW = h @ g['W'].T
        s_src = hW @ g['a_src']
        s_dst = hW @ g['a_dst']
        M = jax.nn.leaky_relu(jnp.max(s_src) + jnp.max(s_dst), 0.2)
        cons = jnp.zeros((160,), _F32).at[0:16].set(M)
        o0, o1, o2 = _gat_pass(_padn(s_src), _padn(s_dst),
                               _padn(hW[:, 0]), _padn(hW[:, 1]),
                               _padn(hW[:, 2]), srcp, dstp, cons)
        h = jnp.stack([o0[:n], o1[:n], o2[:n]], axis=1) + g['b']
        if i < nl - 1:
            h = jax.nn.relu(h)
    p = params['mha']
    q = h @ p['Wq'].T + p['bq']
    k = h @ p['Wk'].T + p['bk']
    v = h @ p['Wv'].T + p['bv']
    outs = []
    for hh in range(3):
        qh, kh, vh = q[:, hh], k[:, hh], v[:, hh]
        kmax, kmin = jnp.max(kh), jnp.min(kh)
        mrow = jnp.where(qh >= 0, qh * kmax, qh * kmin)
        ex = jnp.exp(qh[:, None] * kh[None, :] - mrow[:, None])
        outs.append((ex @ vh) / ex.sum(1))
    h = jnp.stack(outs, axis=1) @ p['Wo'].T + p['bo']
    tc = params['tc']
    u = h @ (tc['Wq'].T @ tc['Wk']) / jnp.sqrt(3.0)
    mt = 3.0 * jnp.max(jnp.abs(u)) * jnp.max(jnp.abs(h))
    cons = jnp.zeros((160,), _F32).at[0:16].set(mt)
    for i in range(3):
        for j in range(3):
            s = 16 * (1 + 3 * i + j)
            cons = cons.at[s:s + 16].set(tc['Wv'][i, j])
    o0, o1, o2 = _tc_pass(_padn(u[:, 0]), _padn(u[:, 1]), _padn(u[:, 2]),
                          _padn(h[:, 0]), _padn(h[:, 1]), _padn(h[:, 2]),
                          srcp, dstp, cons)
    h = (jnp.stack([o0[:n], o1[:n], o2[:n]], axis=1)
         + h @ tc['Ws'].T + tc['b'])
    new_results, rmean = _mlp_tail(h, mask, params['mlp'])
    value = rmean * params['critic']['W'][0, 0] + params['critic']['b'][0]
    return new_results, value


# speedup vs baseline: 146.5919x; 1.5458x over previous
import functools
import jax
import jax.numpy as jnp
from jax import lax
from jax.experimental import pallas as pl
from jax.experimental.pallas import tpu as pltpu
from jax.experimental.pallas import tpu_sc as plsc

NT = 16
NC = 2
NPAD = 10240
ROWS_PT = NPAD // NT
EPAD = 655360
EPT = EPAD // (NC * NT)
ECH = 4096
PADROW = 10200

_mesh = plsc.VectorSubcoreMesh(core_axis_name="c", subcore_axis_name="s",
                               num_cores=NC)

_F32 = jnp.float32


def _zero_accs(a0, a1, a2, ad):
    z = jnp.zeros((16,), _F32)

    def zl(i, c):
        s = pl.ds(i * 16, 16)
        a0[s] = z
        a1[s] = z
        a2[s] = z
        ad[s] = z
        return c

    lax.fori_loop(0, NPAD // 16, zl, 0)


def _reduce_and_emit(cid, w, a0, a1, a2, ad, sh, red, racc, o0, o1, o2, od):
    rb = pl.ds(w * ROWS_PT, ROWS_PT)
    z = jnp.zeros((16,), _F32)

    for priv, oc in ((ad, od), (a0, o0), (a1, o1), (a2, o2)):
        pltpu.sync_copy(priv, sh.at[w])
        plsc.subcore_barrier()

        def zacc(i, c):
            racc[pl.ds(i * 16, 16)] = z
            return c

        def aacc(i, c):
            s = pl.ds(i * 16, 16)
            racc[s] = racc[s] + red[s]
            return c

        lax.fori_loop(0, ROWS_PT // 16, zacc, 0)
        for t in range(NT):
            pltpu.sync_copy(sh.at[t, rb], red)
            lax.fori_loop(0, ROWS_PT // 16, aacc, 0)

        pltpu.sync_copy(racc, oc.at[cid, rb])
        plsc.subcore_barrier()


def _gat_body(ssrc_h, sdst_h, p0_h, p1_h, p2_h, src_h, dst_h, cons_h,
              o0, o1, o2, od,
              ssrc, sdst, p0, p1, p2, a0, a1, a2, ad, ebs, ebd, cbuf,
              red, racc, sh):
    cid = lax.axis_index("c")
    w = lax.axis_index("s")
    pltpu.sync_copy(ssrc_h, ssrc)
    pltpu.sync_copy(sdst_h, sdst)
    pltpu.sync_copy(p0_h, p0)
    pltpu.sync_copy(p1_h, p1)
    pltpu.sync_copy(p2_h, p2)
    pltpu.sync_copy(cons_h, cbuf)
    m = cbuf[pl.ds(0, 16)]
    _zero_accs(a0, a1, a2, ad)
    ebase = (cid * NT + w) * EPT
    for c in range(EPT // ECH):
        pltpu.sync_copy(src_h.at[pl.ds(ebase + c * ECH, ECH)], ebs)
        pltpu.sync_copy(dst_h.at[pl.ds(ebase + c * ECH, ECH)], ebd)

        def el(j, cc):
            s = pl.ds(j * 16, 16)
            si = ebs[s]
            di = ebd[s]
            e = plsc.load_gather(ssrc, [si]) + plsc.load_gather(sdst, [di])
            e = jnp.maximum(e, e * 0.2)
            wt = jnp.exp(e - m)
            plsc.addupdate_scatter(ad, [di], wt)
            plsc.addupdate_scatter(a0, [di], wt * plsc.load_gather(p0, [si]))
            plsc.addupdate_scatter(a1, [di], wt * plsc.load_gather(p1, [si]))
            plsc.addupdate_scatter(a2, [di], wt * plsc.load_gather(p2, [si]))
            return cc

        lax.fori_loop(0, ECH // 16, el, 0)
    _reduce_and_emit(cid, w, a0, a1, a2, ad, sh, red, racc, o0, o1, o2, od)


def _tc_body(u0_h, u1_h, u2_h, h0_h, h1_h, h2_h, src_h, dst_h, cons_h,
             o0, o1, o2, od,
             u0, u1, u2, t0, t1, t2, a0, a1, a2, ad, ebs, ebd, cbuf,
             red, racc, sh):
    cid = lax.axis_index("c")
    w = lax.axis_index("s")
    pltpu.sync_copy(u0_h, u0)
    pltpu.sync_copy(u1_h, u1)
    pltpu.sync_copy(u2_h, u2)
    pltpu.sync_copy(h0_h, t0)
    pltpu.sync_copy(h1_h, t1)
    pltpu.sync_copy(h2_h, t2)
    pltpu.sync_copy(cons_h, cbuf)
    m = cbuf[pl.ds(0, 16)]
    wv = [[cbuf[pl.ds(16 * (1 + 3 * i + j), 16)] for j in range(3)]
          for i in range(3)]
    _zero_accs(a0, a1, a2, ad)
    ebase = (cid * NT + w) * EPT
    for c in range(EPT // ECH):
        pltpu.sync_copy(src_h.at[pl.ds(ebase + c * ECH, ECH)], ebs)
        pltpu.sync_copy(dst_h.at[pl.ds(ebase + c * ECH, ECH)], ebd)

        def el(j, cc):
            s = pl.ds(j * 16, 16)
            si = ebs[s]
            di = ebd[s]
            hg0 = plsc.load_gather(t0, [si])
            hg1 = plsc.load_gather(t1, [si])
            hg2 = plsc.load_gather(t2, [si])
            e = (plsc.load_gather(u0, [di]) * hg0 +
                 plsc.load_gather(u1, [di]) * hg1 +
                 plsc.load_gather(u2, [di]) * hg2)
            wt = jnp.exp(e - m)
            plsc.addupdate_scatter(ad, [di], wt)
            v0 = hg0 * wv[0][0] + hg1 * wv[0][1] + hg2 * wv[0][2]
            v1 = hg0 * wv[1][0] + hg1 * wv[1][1] + hg2 * wv[1][2]
            v2 = hg0 * wv[2][0] + hg1 * wv[2][1] + hg2 * wv[2][2]
            plsc.addupdate_scatter(a0, [di], wt * v0)
            plsc.addupdate_scatter(a1, [di], wt * v1)
            plsc.addupdate_scatter(a2, [di], wt * v2)
            return cc

        lax.fori_loop(0, ECH // 16, el, 0)
    _reduce_and_emit(cid, w, a0, a1, a2, ad, sh, red, racc, o0, o1, o2, od)


def _edge_pass_scratch():
    return [
        pltpu.VMEM((NPAD,), _F32),
        pltpu.VMEM((NPAD,), _F32),
        pltpu.VMEM((NPAD,), _F32),
        pltpu.VMEM((NPAD,), _F32),
        pltpu.VMEM((NPAD,), _F32),
        pltpu.VMEM((NPAD,), _F32),
        pltpu.VMEM((NPAD,), _F32),
        pltpu.VMEM((NPAD,), _F32),
        pltpu.VMEM((NPAD,), _F32),
        pltpu.VMEM((ECH,), jnp.int32),
        pltpu.VMEM((ECH,), jnp.int32),
        pltpu.VMEM((160,), _F32),
        pltpu.VMEM((ROWS_PT,), _F32),
        pltpu.VMEM((ROWS_PT,), _F32),
        pltpu.VMEM_SHARED((NT, NPAD), _F32),
    ]


def _tc_scratch():
    sc = _edge_pass_scratch()
    return sc[:5] + [pltpu.VMEM((NPAD,), _F32)] + sc[5:]


_OUT3 = (jax.ShapeDtypeStruct((NC, NPAD), _F32),) * 4

_gat_pass = functools.partial(
    pl.kernel, out_type=_OUT3, mesh=_mesh,
    compiler_params=pltpu.CompilerParams(needs_layout_passes=False),
    scratch_types=_edge_pass_scratch())(_gat_body)

_tc_pass = functools.partial(
    pl.kernel, out_type=_OUT3, mesh=_mesh,
    compiler_params=pltpu.CompilerParams(needs_layout_passes=False),
    scratch_types=_tc_scratch())(_tc_body)


_BQ = 1024
_NKC = NPAD // _BQ


def _mha_body(q_ref, mr_ref, k_ref, vm_ref, o_ref):
    qb = q_ref[...]
    mb = mr_ref[...]
    kk = k_ref[...]
    vm = vm_ref[...]
    rows = []
    for h in range(3):
        num = jnp.zeros((_BQ,), _F32)
        den = jnp.zeros((_BQ,), _F32)
        for t in range(_NKC):
            ks = kk[h, t * _BQ:(t + 1) * _BQ]
            ex = jnp.exp(qb[h][:, None] * ks[None, :] - mb[h][:, None])
            nd = jnp.dot(ex, vm[h, t * _BQ:(t + 1) * _BQ, :],
                         preferred_element_type=_F32)
            num = num + nd[:, 0]
            den = den + nd[:, 1]
        rows.append(num / den)
    o_ref[...] = jnp.stack(rows, axis=0)


def _mha(h, p):
    n = h.shape[0]
    q = h @ p['Wq'].T + p['bq']
    k = h @ p['Wk'].T + p['bk']
    v = h @ p['Wv'].T + p['bv']
    kmax = jnp.max(k, axis=0)
    kmin = jnp.min(k, axis=0)
    mrow = jnp.where(q >= 0, q * kmax[None, :], q * kmin[None, :])
    pad = ((0, NPAD - n), (0, 0))
    qT = jnp.pad(q, pad).T
    mrT = jnp.pad(mrow, pad).T
    kT = jnp.pad(k, pad).T
    vT = jnp.pad(v, pad).T
    msk = jnp.pad(jnp.ones((n,), _F32), (0, NPAD - n))
    vm3 = jnp.stack([vT, jnp.broadcast_to(msk[None, :], (3, NPAD))], axis=2)
    o = pl.pallas_call(
        _mha_body,
        grid=(NPAD // _BQ,),
        in_specs=[
            pl.BlockSpec((3, _BQ), lambda i: (0, i)),
            pl.BlockSpec((3, _BQ), lambda i: (0, i)),
            pl.BlockSpec((3, NPAD), lambda i: (0, 0)),
            pl.BlockSpec((3, NPAD, 2), lambda i: (0, 0, 0)),
        ],
        out_specs=pl.BlockSpec((3, _BQ), lambda i: (0, i)),
        out_shape=jax.ShapeDtypeStruct((3, NPAD), _F32),
    )(qT, mrT, kT, vm3)
    return o[:, :n].T @ p['Wo'].T + p['bo']


def _tail_kernel(h_ref, mask_ref, w1_ref, b1_ref, w2_ref, b2_ref, w3_ref,
                 b3_ref, out_ref, val_ref):
    h = h_ref[...]
    z = jnp.maximum(jnp.dot(h, w1_ref[...], preferred_element_type=_F32)
                    + b1_ref[...][None, :], 0.0)
    z = jnp.maximum(jnp.dot(z, w2_ref[...], preferred_element_type=_F32)
                    + b2_ref[...][None, :], 0.0)
    res = jnp.dot(z, w3_ref[...], preferred_element_type=_F32)
    res = res + b3_ref[0]
    out_ref[...] = res[:, 0] * mask_ref[...]
    val_ref[...] = jnp.mean(res)[None, None]


def _mlp_tail(h, mask, m):
    n = h.shape[0]
    w1 = m['W1'][:, :3].T
    out, rmean = pl.pallas_call(
        _tail_kernel,
        out_shape=(
            jax.ShapeDtypeStruct((n,), _F32),
            jax.ShapeDtypeStruct((1, 1), _F32),
        ),
    )(h, mask, w1, m['b1'], m['W2'].T, m['b2'], m['W3'].T, m['b3'])
    return out, rmean[0, 0]


def _padn(v):
    return jnp.pad(v, (0, NPAD - v.shape[0]))


def kernel(x, edge_index, mask, params):
    n = x.shape[0]
    loop = jnp.arange(n, dtype=jnp.int32)
    nreal = edge_index.shape[1] + n
    npadE = EPAD - nreal
    srcp = jnp.concatenate([edge_index[0].astype(jnp.int32), loop,
                            jnp.zeros((npadE,), jnp.int32)])
    dstp = jnp.concatenate([edge_index[1].astype(jnp.int32), loop,
                            jnp.full((npadE,), PADROW, jnp.int32)])
    h = x
    nl = len(params['gat'])
    for i in range(nl):
        g = params['gat'][i]
        hW = h @ g['W'].T
        s_src = hW @ g['a_src']
        s_dst = hW @ g['a_dst']
        M = jax.nn.leaky_relu(jnp.max(s_src) + jnp.max(s_dst), 0.2)
        cons = jnp.zeros((160,), _F32).at[0:16].set(M)
        o0, o1, o2, od = _gat_pass(_padn(s_src), _padn(s_dst),
                                   _padn(hW[:, 0]), _padn(hW[:, 1]),
                                   _padn(hW[:, 2]), srcp, dstp, cons)
        den = jnp.maximum(od[0] + od[1], 1e-30)[:n, None]
        h = (jnp.stack([(o0[0] + o0[1])[:n], (o1[0] + o1[1])[:n],
                        (o2[0] + o2[1])[:n]], axis=1) / den + g['b'])
        if i < nl - 1:
            h = jax.nn.relu(h)
    h = _mha(h, params['mha'])
    tc = params['tc']
    u = h @ (tc['Wq'].T @ tc['Wk']) / jnp.sqrt(3.0)
    mt = 3.0 * jnp.max(jnp.abs(u)) * jnp.max(jnp.abs(h))
    cons = jnp.zeros((160,), _F32).at[0:16].set(mt)
    for i in range(3):
        for j in range(3):
            s = 16 * (1 + 3 * i + j)
            cons = cons.at[s:s + 16].set(tc['Wv'][i, j])
    o0, o1, o2, od = _tc_pass(_padn(u[:, 0]), _padn(u[:, 1]), _padn(u[:, 2]),
                              _padn(h[:, 0]), _padn(h[:, 1]), _padn(h[:, 2]),
                              srcp, dstp, cons)
    den = jnp.maximum(od[0] + od[1], 1e-30)[:n, None]
    h = (jnp.stack([(o0[0] + o0[1])[:n], (o1[0] + o1[1])[:n],
                    (o2[0] + o2[1])[:n]], axis=1) / den
         + h @ tc['Ws'].T + tc['b'])
    new_results, rmean = _mlp_tail(h, mask, params['mlp'])
    value = rmean * params['critic']['W'][0, 0] + params['critic']['b'][0]
    return new_results, value
